# baseline (speedup 1.0000x reference)
import functools

import jax
import jax.numpy as jnp
from jax.experimental import pallas as pl
from jax.experimental.pallas import tpu as pltpu

DIM = 256
QK = 256
HEADS = 8
TOPK = 2
WV = 2
SCALE = QK ** -0.5
CH = QK // HEADS
CV = DIM // HEADS
W2 = 128
NWIN = 32


def _qkv_kernel(x_ref, w_ref, b_ref, kt_ref, vq_ref, qwin_ref, kwin_ref):
    acc = jnp.dot(x_ref[...], w_ref[...], preferred_element_type=jnp.float32)
    acc = acc + b_ref[...]
    kt_ref[...] = acc[:, :QK].astype(jnp.bfloat16).T
    vq_ref[...] = acc[:, QK:].astype(jnp.bfloat16)
    for w in range(acc.shape[0] // W2):
        blk = acc[w * W2:(w + 1) * W2, :]
        qwin_ref[w, :, :] = jnp.mean(blk[:, 2 * QK:], axis=0, keepdims=True)
        kwin_ref[w, :, :] = jnp.mean(blk[:, :QK], axis=0, keepdims=True)


def _router_kernel(qwin_ref, kwin_ref, idx_ref):
    qw = qwin_ref[...].reshape(-1, QK)
    kw = kwin_ref[...].reshape(-1, QK)
    for n in range(2):
        q = qw[n * NWIN:(n + 1) * NWIN, :]
        k = kw[n * NWIN:(n + 1) * NWIN, :]
        logits = jnp.dot(q, k.T, preferred_element_type=jnp.float32)
        cols = jax.lax.broadcasted_iota(jnp.int32, logits.shape, 1)
        big = jnp.int32(NWIN)
        m0 = jnp.max(logits, axis=-1, keepdims=True)
        i0 = jnp.min(jnp.where(logits == m0, cols, big), axis=-1,
                     keepdims=True)
        masked = jnp.where(cols == i0, -jnp.inf, logits)
        m1 = jnp.max(masked, axis=-1, keepdims=True)
        i1 = jnp.min(jnp.where(masked == m1, cols, big), axis=-1,
                     keepdims=True)
        base = jnp.int32(n * NWIN)
        idx_ref[n * NWIN:(n + 1) * NWIN, :] = (
            jnp.concatenate([i0, i1], axis=-1) + base)


def _lepe_kernel(v_ref, w_ref, out_ref):
    v = v_ref[...].astype(jnp.float32).reshape(64, 64, DIM)
    zr = jnp.zeros((1, 64, DIM), jnp.float32)
    zc = jnp.zeros((64, 1, DIM), jnp.float32)

    def shift(a, dy, dx):
        if dy == -1:
            a = jnp.concatenate([zr, a[:-1]], axis=0)
        elif dy == 1:
            a = jnp.concatenate([a[1:], zr], axis=0)
        if dx == -1:
            a = jnp.concatenate([zc, a[:, :-1]], axis=1)
        elif dx == 1:
            a = jnp.concatenate([a[:, 1:], zc], axis=1)
        return a

    acc = v * w_ref[4, :]
    for dy in (-1, 0, 1):
        for dx in (-1, 0, 1):
            if dy == 0 and dx == 0:
                continue
            acc = acc + shift(v, dy, dx) * w_ref[3 * (dy + 1) + (dx + 1), :]
    out_ref[...] = acc.reshape(64 * 64, DIM)


WPS = 1


def _attn_kernel(idx_ref, q_ref, *refs):
    del idx_ref
    gather = refs[:4 * WPS]
    lepe_ref, wo_ref, bo_ref, out_ref = refs[4 * WPS:]
    for w in range(WPS):
        kta_ref, ktb_ref, va_ref, vb_ref = gather[4 * w:4 * w + 4]
        rows = slice(w * W2, (w + 1) * W2)
        outs = []
        for h in range(HEADS):
            sl = slice(h * CH, (h + 1) * CH)
            qh = q_ref[rows, sl]
            ea = jnp.exp(jnp.dot(qh, kta_ref[sl, :],
                                 preferred_element_type=jnp.float32))
            eb = jnp.exp(jnp.dot(qh, ktb_ref[sl, :],
                                 preferred_element_type=jnp.float32))
            s = (jnp.sum(ea, axis=-1, keepdims=True)
                 + jnp.sum(eb, axis=-1, keepdims=True))
            pv = (jnp.dot(ea.astype(jnp.bfloat16), va_ref[:, sl],
                          preferred_element_type=jnp.float32)
                  + jnp.dot(eb.astype(jnp.bfloat16), vb_ref[:, sl],
                            preferred_element_type=jnp.float32))
            outs.append(pv * (1.0 / s))
        attn = jnp.concatenate(outs, axis=-1)
        acc = attn + lepe_ref[rows, :]
        out_ref[rows, :] = (
            jnp.dot(acc, wo_ref[...], preferred_element_type=jnp.float32)
            + bo_ref[...])


@functools.partial(jax.jit, static_argnames=())
def _forward_impl(x, W_qkv, b_qkv, W_o, b_o, lepe_w, lepe_b):
    N, C, H, W = x.shape
    P = N * H * W
    nwin_total = N * NWIN
    xp = jnp.transpose(x, (0, 2, 3, 1)).reshape(P, C)
    Wr = jnp.concatenate([W_qkv[:, QK:2 * QK], W_qkv[:, 2 * QK:],
                          W_qkv[:, :QK] * SCALE], axis=1)
    br = jnp.concatenate([b_qkv[QK:2 * QK], b_qkv[2 * QK:],
                          b_qkv[:QK] * SCALE]).reshape(1, -1)

    QT = 4
    kt, vq, qwin, kwin = pl.pallas_call(
        _qkv_kernel,
        grid=(nwin_total // QT,),
        in_specs=[
            pl.BlockSpec((QT * W2, C), lambda i: (i, 0)),
            pl.BlockSpec((C, 2 * QK + DIM), lambda i: (0, 0)),
            pl.BlockSpec((1, 2 * QK + DIM), lambda i: (0, 0)),
        ],
        out_specs=[
            pl.BlockSpec((QK, QT * W2), lambda i: (0, i)),
            pl.BlockSpec((QT * W2, QK + DIM), lambda i: (i, 0)),
            pl.BlockSpec((QT, 1, QK), lambda i: (i, 0, 0)),
            pl.BlockSpec((QT, 1, QK), lambda i: (i, 0, 0)),
        ],
        out_shape=[
            jax.ShapeDtypeStruct((QK, P), jnp.bfloat16),
            jax.ShapeDtypeStruct((P, QK + DIM), jnp.bfloat16),
            jax.ShapeDtypeStruct((nwin_total, 1, QK), jnp.float32),
            jax.ShapeDtypeStruct((nwin_total, 1, QK), jnp.float32),
        ],
    )(xp, Wr, br)

    idx2d = pl.pallas_call(
        _router_kernel,
        grid=(1,),
        in_specs=[
            pl.BlockSpec((nwin_total, 1, QK), lambda i: (0, 0, 0)),
            pl.BlockSpec((nwin_total, 1, QK), lambda i: (0, 0, 0)),
        ],
        out_specs=pl.BlockSpec((nwin_total, TOPK), lambda i: (0, 0)),
        out_shape=jax.ShapeDtypeStruct((nwin_total, TOPK), jnp.int32),
    )(qwin, kwin)

    lw = jnp.transpose(lepe_w.reshape(DIM, 3, 3), (1, 2, 0)).reshape(9, DIM)
    lepe = pl.pallas_call(
        _lepe_kernel,
        grid=(N,),
        in_specs=[
            pl.BlockSpec((H * W, DIM), lambda n: (n, 0)),
            pl.BlockSpec((9, DIM), lambda n: (0, 0)),
        ],
        out_specs=pl.BlockSpec((H * W, DIM), lambda n: (n, 0)),
        out_shape=jax.ShapeDtypeStruct((P, DIM), jnp.float32),
    )(vq, lw)
    b_eff = (b_o + lepe_b @ W_o).reshape(1, -1)

    gather_specs = []
    gather_args = []
    for w in range(WPS):
        gather_specs += [
            pl.BlockSpec((QK, W2), lambda i, s, w=w: (0, s[WPS * i + w, 0])),
            pl.BlockSpec((QK, W2), lambda i, s, w=w: (0, s[WPS * i + w, 1])),
            pl.BlockSpec((W2, DIM), lambda i, s, w=w: (s[WPS * i + w, 0], 0)),
            pl.BlockSpec((W2, DIM), lambda i, s, w=w: (s[WPS * i + w, 1], 0)),
        ]
        gather_args += [kt, kt, vq, vq]
    grid_spec = pltpu.PrefetchScalarGridSpec(
        num_scalar_prefetch=1,
        grid=(nwin_total // WPS,),
        in_specs=(
            [pl.BlockSpec((WPS * W2, QK), lambda i, s: (i, 1))]
            + gather_specs
            + [pl.BlockSpec((WPS * W2, DIM), lambda i, s: (i, 0)),
               pl.BlockSpec((DIM, DIM), lambda i, s: (0, 0)),
               pl.BlockSpec((1, DIM), lambda i, s: (0, 0))]
        ),
        out_specs=pl.BlockSpec((WPS * W2, DIM), lambda i, s: (i, 0)),
    )
    out = pl.pallas_call(
        _attn_kernel,
        grid_spec=grid_spec,
        out_shape=jax.ShapeDtypeStruct((P, DIM), jnp.float32),
    )(idx2d, vq, *gather_args, lepe, W_o, b_eff)

    return out.reshape(N, H, W, DIM)


def kernel(x, W_qkv, b_qkv, W_o, b_o, lepe_w, lepe_b):
    return _forward_impl(x, W_qkv, b_qkv, W_o, b_o, lepe_w, lepe_b)


# speedup vs baseline: 1.1473x; 1.1473x over previous
import functools

import jax
import jax.numpy as jnp
from jax.experimental import pallas as pl
from jax.experimental.pallas import tpu as pltpu

DIM = 256
QK = 256
HEADS = 8
TOPK = 2
WV = 2
SCALE = QK ** -0.5
CH = QK // HEADS
CV = DIM // HEADS
W2 = 128
NWIN = 32


def _qkv_kernel(x_ref, w_ref, b_ref, kt_ref, vq_ref, qwin_ref, kwin_ref):
    acc = jnp.dot(x_ref[...], w_ref[...], preferred_element_type=jnp.float32)
    acc = acc + b_ref[...]
    kt_ref[...] = acc[:, :QK].astype(jnp.bfloat16).T
    vq_ref[...] = acc[:, QK:].astype(jnp.bfloat16)
    for w in range(acc.shape[0] // W2):
        blk = acc[w * W2:(w + 1) * W2, :]
        qwin_ref[w, :, :] = jnp.mean(blk[:, 2 * QK:], axis=0, keepdims=True)
        kwin_ref[w, :, :] = jnp.mean(blk[:, :QK], axis=0, keepdims=True)


def _router_kernel(qwin_ref, kwin_ref, idx_ref):
    qw = qwin_ref[...].reshape(-1, QK)
    kw = kwin_ref[...].reshape(-1, QK)
    for n in range(2):
        q = qw[n * NWIN:(n + 1) * NWIN, :]
        k = kw[n * NWIN:(n + 1) * NWIN, :]
        logits = jnp.dot(q, k.T, preferred_element_type=jnp.float32)
        cols = jax.lax.broadcasted_iota(jnp.int32, logits.shape, 1)
        big = jnp.int32(NWIN)
        m0 = jnp.max(logits, axis=-1, keepdims=True)
        i0 = jnp.min(jnp.where(logits == m0, cols, big), axis=-1,
                     keepdims=True)
        masked = jnp.where(cols == i0, -jnp.inf, logits)
        m1 = jnp.max(masked, axis=-1, keepdims=True)
        i1 = jnp.min(jnp.where(masked == m1, cols, big), axis=-1,
                     keepdims=True)
        base = jnp.int32(n * NWIN)
        idx_ref[n * NWIN:(n + 1) * NWIN, :] = (
            jnp.concatenate([i0, i1], axis=-1) + base)


def _lepe_kernel(v_ref, w_ref, out_ref):
    v = v_ref[...].astype(jnp.float32).reshape(64, 64, DIM)
    zr = jnp.zeros((1, 64, DIM), jnp.float32)
    zc = jnp.zeros((64, 1, DIM), jnp.float32)

    def shift(a, dy, dx):
        if dy == -1:
            a = jnp.concatenate([zr, a[:-1]], axis=0)
        elif dy == 1:
            a = jnp.concatenate([a[1:], zr], axis=0)
        if dx == -1:
            a = jnp.concatenate([zc, a[:, :-1]], axis=1)
        elif dx == 1:
            a = jnp.concatenate([a[:, 1:], zc], axis=1)
        return a

    acc = v * w_ref[4, :]
    for dy in (-1, 0, 1):
        for dx in (-1, 0, 1):
            if dy == 0 and dx == 0:
                continue
            acc = acc + shift(v, dy, dx) * w_ref[3 * (dy + 1) + (dx + 1), :]
    out_ref[...] = acc.reshape(64 * 64, DIM)


WPS = 2


def _attn_kernel(idx_ref, q_ref, *refs):
    del idx_ref
    gather = refs[:4 * WPS]
    lepe_ref, wo_ref, bo_ref, out_ref = refs[4 * WPS:]
    for w in range(WPS):
        kta_ref, ktb_ref, va_ref, vb_ref = gather[4 * w:4 * w + 4]
        rows = slice(w * W2, (w + 1) * W2)
        outs = []
        for h in range(HEADS):
            sl = slice(h * CH, (h + 1) * CH)
            qh = q_ref[rows, sl]
            ea = jnp.exp(jnp.dot(qh, kta_ref[sl, :],
                                 preferred_element_type=jnp.float32))
            eb = jnp.exp(jnp.dot(qh, ktb_ref[sl, :],
                                 preferred_element_type=jnp.float32))
            s = (jnp.sum(ea, axis=-1, keepdims=True)
                 + jnp.sum(eb, axis=-1, keepdims=True))
            pv = (jnp.dot(ea.astype(jnp.bfloat16), va_ref[:, sl],
                          preferred_element_type=jnp.float32)
                  + jnp.dot(eb.astype(jnp.bfloat16), vb_ref[:, sl],
                            preferred_element_type=jnp.float32))
            outs.append(pv * (1.0 / s))
        attn = jnp.concatenate(outs, axis=-1)
        acc = attn + lepe_ref[rows, :]
        out_ref[rows, :] = (
            jnp.dot(acc, wo_ref[...], preferred_element_type=jnp.float32)
            + bo_ref[...])


@functools.partial(jax.jit, static_argnames=())
def _forward_impl(x, W_qkv, b_qkv, W_o, b_o, lepe_w, lepe_b):
    N, C, H, W = x.shape
    P = N * H * W
    nwin_total = N * NWIN
    xp = jnp.transpose(x, (0, 2, 3, 1)).reshape(P, C)
    Wr = jnp.concatenate([W_qkv[:, QK:2 * QK], W_qkv[:, 2 * QK:],
                          W_qkv[:, :QK] * SCALE], axis=1)
    br = jnp.concatenate([b_qkv[QK:2 * QK], b_qkv[2 * QK:],
                          b_qkv[:QK] * SCALE]).reshape(1, -1)

    QT = 8
    kt, vq, qwin, kwin = pl.pallas_call(
        _qkv_kernel,
        grid=(nwin_total // QT,),
        in_specs=[
            pl.BlockSpec((QT * W2, C), lambda i: (i, 0)),
            pl.BlockSpec((C, 2 * QK + DIM), lambda i: (0, 0)),
            pl.BlockSpec((1, 2 * QK + DIM), lambda i: (0, 0)),
        ],
        out_specs=[
            pl.BlockSpec((QK, QT * W2), lambda i: (0, i)),
            pl.BlockSpec((QT * W2, QK + DIM), lambda i: (i, 0)),
            pl.BlockSpec((QT, 1, QK), lambda i: (i, 0, 0)),
            pl.BlockSpec((QT, 1, QK), lambda i: (i, 0, 0)),
        ],
        out_shape=[
            jax.ShapeDtypeStruct((QK, P), jnp.bfloat16),
            jax.ShapeDtypeStruct((P, QK + DIM), jnp.bfloat16),
            jax.ShapeDtypeStruct((nwin_total, 1, QK), jnp.float32),
            jax.ShapeDtypeStruct((nwin_total, 1, QK), jnp.float32),
        ],
    )(xp, Wr, br)

    idx2d = pl.pallas_call(
        _router_kernel,
        grid=(1,),
        in_specs=[
            pl.BlockSpec((nwin_total, 1, QK), lambda i: (0, 0, 0)),
            pl.BlockSpec((nwin_total, 1, QK), lambda i: (0, 0, 0)),
        ],
        out_specs=pl.BlockSpec((nwin_total, TOPK), lambda i: (0, 0)),
        out_shape=jax.ShapeDtypeStruct((nwin_total, TOPK), jnp.int32),
    )(qwin, kwin)

    lw = jnp.transpose(lepe_w.reshape(DIM, 3, 3), (1, 2, 0)).reshape(9, DIM)
    lepe = pl.pallas_call(
        _lepe_kernel,
        grid=(N,),
        in_specs=[
            pl.BlockSpec((H * W, DIM), lambda n: (n, 0)),
            pl.BlockSpec((9, DIM), lambda n: (0, 0)),
        ],
        out_specs=pl.BlockSpec((H * W, DIM), lambda n: (n, 0)),
        out_shape=jax.ShapeDtypeStruct((P, DIM), jnp.float32),
    )(vq, lw)
    b_eff = (b_o + lepe_b @ W_o).reshape(1, -1)

    gather_specs = []
    gather_args = []
    for w in range(WPS):
        gather_specs += [
            pl.BlockSpec((QK, W2), lambda i, s, w=w: (0, s[WPS * i + w, 0])),
            pl.BlockSpec((QK, W2), lambda i, s, w=w: (0, s[WPS * i + w, 1])),
            pl.BlockSpec((W2, DIM), lambda i, s, w=w: (s[WPS * i + w, 0], 0)),
            pl.BlockSpec((W2, DIM), lambda i, s, w=w: (s[WPS * i + w, 1], 0)),
        ]
        gather_args += [kt, kt, vq, vq]
    grid_spec = pltpu.PrefetchScalarGridSpec(
        num_scalar_prefetch=1,
        grid=(nwin_total // WPS,),
        in_specs=(
            [pl.BlockSpec((WPS * W2, QK), lambda i, s: (i, 1))]
            + gather_specs
            + [pl.BlockSpec((WPS * W2, DIM), lambda i, s: (i, 0)),
               pl.BlockSpec((DIM, DIM), lambda i, s: (0, 0)),
               pl.BlockSpec((1, DIM), lambda i, s: (0, 0))]
        ),
        out_specs=pl.BlockSpec((WPS * W2, DIM), lambda i, s: (i, 0)),
    )
    out = pl.pallas_call(
        _attn_kernel,
        grid_spec=grid_spec,
        out_shape=jax.ShapeDtypeStruct((P, DIM), jnp.float32),
    )(idx2d, vq, *gather_args, lepe, W_o, b_eff)

    return out.reshape(N, H, W, DIM)


def kernel(x, W_qkv, b_qkv, W_o, b_o, lepe_w, lepe_b):
    return _forward_impl(x, W_qkv, b_qkv, W_o, b_o, lepe_w, lepe_b)


# speedup vs baseline: 1.1735x; 1.0228x over previous
import functools

import jax
import jax.numpy as jnp
from jax.experimental import pallas as pl
from jax.experimental.pallas import tpu as pltpu

DIM = 256
QK = 256
HEADS = 8
TOPK = 2
WV = 2
SCALE = QK ** -0.5
CH = QK // HEADS
CV = DIM // HEADS
W2 = 128
NWIN = 32


def _qkv_kernel(x_ref, w_ref, b_ref, kt_ref, vq_ref, qwin_ref, kwin_ref):
    acc = jnp.dot(x_ref[...], w_ref[...], preferred_element_type=jnp.float32)
    acc = acc + b_ref[...]
    kt_ref[...] = acc[:, :QK].astype(jnp.bfloat16).T
    vq_ref[...] = acc[:, QK:].astype(jnp.bfloat16)
    for w in range(acc.shape[0] // W2):
        blk = acc[w * W2:(w + 1) * W2, :]
        qwin_ref[w, :, :] = jnp.mean(blk[:, 2 * QK:], axis=0, keepdims=True)
        kwin_ref[w, :, :] = jnp.mean(blk[:, :QK], axis=0, keepdims=True)


def _router_kernel(qwin_ref, kwin_ref, idx_ref):
    qw = qwin_ref[...].reshape(-1, QK)
    kw = kwin_ref[...].reshape(-1, QK)
    for n in range(2):
        q = qw[n * NWIN:(n + 1) * NWIN, :]
        k = kw[n * NWIN:(n + 1) * NWIN, :]
        logits = jnp.dot(q, k.T, preferred_element_type=jnp.float32)
        cols = jax.lax.broadcasted_iota(jnp.int32, logits.shape, 1)
        big = jnp.int32(NWIN)
        m0 = jnp.max(logits, axis=-1, keepdims=True)
        i0 = jnp.min(jnp.where(logits == m0, cols, big), axis=-1,
                     keepdims=True)
        masked = jnp.where(cols == i0, -jnp.inf, logits)
        m1 = jnp.max(masked, axis=-1, keepdims=True)
        i1 = jnp.min(jnp.where(masked == m1, cols, big), axis=-1,
                     keepdims=True)
        base = jnp.int32(n * NWIN)
        idx_ref[n * NWIN:(n + 1) * NWIN, :] = (
            jnp.concatenate([i0, i1], axis=-1) + base)


def _lepe_kernel(v_ref, w_ref, out_ref):
    v = v_ref[...].astype(jnp.float32).reshape(64, 64, DIM)
    zr = jnp.zeros((1, 64, DIM), jnp.float32)
    zc = jnp.zeros((64, 1, DIM), jnp.float32)

    def shift(a, dy, dx):
        if dy == -1:
            a = jnp.concatenate([zr, a[:-1]], axis=0)
        elif dy == 1:
            a = jnp.concatenate([a[1:], zr], axis=0)
        if dx == -1:
            a = jnp.concatenate([zc, a[:, :-1]], axis=1)
        elif dx == 1:
            a = jnp.concatenate([a[:, 1:], zc], axis=1)
        return a

    acc = v * w_ref[4, :]
    for dy in (-1, 0, 1):
        for dx in (-1, 0, 1):
            if dy == 0 and dx == 0:
                continue
            acc = acc + shift(v, dy, dx) * w_ref[3 * (dy + 1) + (dx + 1), :]
    out_ref[...] = acc.reshape(64 * 64, DIM)


WPS = 2


def _attn_kernel(idx_ref, q_ref, *refs):
    del idx_ref
    gather = refs[:4 * WPS]
    lepe_ref, wo_ref, bo_ref, out_ref = refs[4 * WPS:]
    for w in range(WPS):
        kta_ref, ktb_ref, va_ref, vb_ref = gather[4 * w:4 * w + 4]
        rows = slice(w * W2, (w + 1) * W2)
        outs = []
        for h in range(HEADS):
            sl = slice(h * CH, (h + 1) * CH)
            qh = q_ref[rows, sl]
            ea = jnp.exp(jnp.dot(qh, kta_ref[sl, :],
                                 preferred_element_type=jnp.float32))
            eb = jnp.exp(jnp.dot(qh, ktb_ref[sl, :],
                                 preferred_element_type=jnp.float32))
            s = (jnp.sum(ea, axis=-1, keepdims=True)
                 + jnp.sum(eb, axis=-1, keepdims=True))
            pv = (jnp.dot(ea.astype(jnp.bfloat16), va_ref[:, sl],
                          preferred_element_type=jnp.float32)
                  + jnp.dot(eb.astype(jnp.bfloat16), vb_ref[:, sl],
                            preferred_element_type=jnp.float32))
            outs.append(pv * (1.0 / s))
        attn = jnp.concatenate(outs, axis=-1)
        acc = attn + lepe_ref[rows, :]
        out_ref[rows, :] = (
            jnp.dot(acc, wo_ref[...], preferred_element_type=jnp.float32)
            + bo_ref[...])


@functools.partial(jax.jit, static_argnames=())
def _forward_impl(x, W_qkv, b_qkv, W_o, b_o, lepe_w, lepe_b):
    N, C, H, W = x.shape
    P = N * H * W
    nwin_total = N * NWIN
    xp = jnp.transpose(x, (0, 2, 3, 1)).reshape(P, C)
    Wr = jnp.concatenate([W_qkv[:, QK:2 * QK], W_qkv[:, 2 * QK:],
                          W_qkv[:, :QK] * SCALE], axis=1)
    br = jnp.concatenate([b_qkv[QK:2 * QK], b_qkv[2 * QK:],
                          b_qkv[:QK] * SCALE]).reshape(1, -1)

    QT = 16
    kt, vq, qwin, kwin = pl.pallas_call(
        _qkv_kernel,
        grid=(nwin_total // QT,),
        in_specs=[
            pl.BlockSpec((QT * W2, C), lambda i: (i, 0)),
            pl.BlockSpec((C, 2 * QK + DIM), lambda i: (0, 0)),
            pl.BlockSpec((1, 2 * QK + DIM), lambda i: (0, 0)),
        ],
        out_specs=[
            pl.BlockSpec((QK, QT * W2), lambda i: (0, i)),
            pl.BlockSpec((QT * W2, QK + DIM), lambda i: (i, 0)),
            pl.BlockSpec((QT, 1, QK), lambda i: (i, 0, 0)),
            pl.BlockSpec((QT, 1, QK), lambda i: (i, 0, 0)),
        ],
        out_shape=[
            jax.ShapeDtypeStruct((QK, P), jnp.bfloat16),
            jax.ShapeDtypeStruct((P, QK + DIM), jnp.bfloat16),
            jax.ShapeDtypeStruct((nwin_total, 1, QK), jnp.float32),
            jax.ShapeDtypeStruct((nwin_total, 1, QK), jnp.float32),
        ],
    )(xp, Wr, br)

    idx2d = pl.pallas_call(
        _router_kernel,
        grid=(1,),
        in_specs=[
            pl.BlockSpec((nwin_total, 1, QK), lambda i: (0, 0, 0)),
            pl.BlockSpec((nwin_total, 1, QK), lambda i: (0, 0, 0)),
        ],
        out_specs=pl.BlockSpec((nwin_total, TOPK), lambda i: (0, 0)),
        out_shape=jax.ShapeDtypeStruct((nwin_total, TOPK), jnp.int32),
    )(qwin, kwin)

    lw = jnp.transpose(lepe_w.reshape(DIM, 3, 3), (1, 2, 0)).reshape(9, DIM)
    lepe = pl.pallas_call(
        _lepe_kernel,
        grid=(N,),
        in_specs=[
            pl.BlockSpec((H * W, DIM), lambda n: (n, 0)),
            pl.BlockSpec((9, DIM), lambda n: (0, 0)),
        ],
        out_specs=pl.BlockSpec((H * W, DIM), lambda n: (n, 0)),
        out_shape=jax.ShapeDtypeStruct((P, DIM), jnp.float32),
    )(vq, lw)
    b_eff = (b_o + lepe_b @ W_o).reshape(1, -1)

    gather_specs = []
    gather_args = []
    for w in range(WPS):
        gather_specs += [
            pl.BlockSpec((QK, W2), lambda i, s, w=w: (0, s[WPS * i + w, 0])),
            pl.BlockSpec((QK, W2), lambda i, s, w=w: (0, s[WPS * i + w, 1])),
            pl.BlockSpec((W2, DIM), lambda i, s, w=w: (s[WPS * i + w, 0], 0)),
            pl.BlockSpec((W2, DIM), lambda i, s, w=w: (s[WPS * i + w, 1], 0)),
        ]
        gather_args += [kt, kt, vq, vq]
    grid_spec = pltpu.PrefetchScalarGridSpec(
        num_scalar_prefetch=1,
        grid=(nwin_total // WPS,),
        in_specs=(
            [pl.BlockSpec((WPS * W2, QK), lambda i, s: (i, 1))]
            + gather_specs
            + [pl.BlockSpec((WPS * W2, DIM), lambda i, s: (i, 0)),
               pl.BlockSpec((DIM, DIM), lambda i, s: (0, 0)),
               pl.BlockSpec((1, DIM), lambda i, s: (0, 0))]
        ),
        out_specs=pl.BlockSpec((WPS * W2, DIM), lambda i, s: (i, 0)),
    )
    out = pl.pallas_call(
        _attn_kernel,
        grid_spec=grid_spec,
        out_shape=jax.ShapeDtypeStruct((P, DIM), jnp.float32),
    )(idx2d, vq, *gather_args, lepe, W_o, b_eff)

    return out.reshape(N, H, W, DIM)


def kernel(x, W_qkv, b_qkv, W_o, b_o, lepe_w, lepe_b):
    return _forward_impl(x, W_qkv, b_qkv, W_o, b_o, lepe_w, lepe_b)
